# baseline (speedup 1.0000x reference)
import functools

import jax
import jax.numpy as jnp
from jax import lax
from jax.experimental import pallas as pl
from jax.experimental.pallas import tpu as pltpu
from jax.experimental.pallas import tpu_sc as plsc

VOCAB = 1000000
BATCH = 16384
WIN = 5
EMB = 50
EMBP = 128

NC = 2
NS = 16
NW = NC * NS
CHUNK = 128
TOTAL = BATCH * WIN
PER_W = TOTAL // NW
NCHUNK = PER_W // CHUNK
NPASS = 10
NBUF = NCHUNK // NPASS


def _make_sc_gather():
    mesh = plsc.VectorSubcoreMesh(core_axis_name="c", subcore_axis_name="s")

    @functools.partial(
        pl.kernel,
        mesh=mesh,
        compiler_params=pltpu.CompilerParams(use_tc_tiling_on_sc=False),
        out_type=jax.ShapeDtypeStruct((NW, NCHUNK, CHUNK, EMBP), jnp.bfloat16),
        scratch_types=[
            pltpu.VMEM((NCHUNK, CHUNK), jnp.int32),
            pltpu.VMEM((NBUF, CHUNK, EMBP), jnp.bfloat16),
            pltpu.VMEM((NBUF, CHUNK, EMBP), jnp.bfloat16),
            pltpu.SemaphoreType.DMA,
            pltpu.SemaphoreType.DMA,
        ],
    )
    def gather_k(table_hbm, idx_hbm, out_hbm, idx_v, rows_a, rows_b, sem_a,
                 sem_b):
        wid = lax.axis_index("s") * NC + lax.axis_index("c")
        pltpu.sync_copy(idx_hbm.at[wid], idx_v)
        bufs = (rows_a, rows_b)
        sems = (sem_a, sem_b)

        def fire(p):
            buf, sem = bufs[p % 2], sems[p % 2]
            return [pltpu.async_copy(table_hbm.at[idx_v.at[p * NBUF + j]],
                                     buf.at[j], sem)
                    for j in range(NBUF)]

        def drain(p, copies):
            for c in copies:
                c.wait()
            pltpu.sync_copy(bufs[p % 2],
                            out_hbm.at[wid, pl.ds(p * NBUF, NBUF)])

        pend = [fire(0), fire(1)]
        for p in range(NPASS):
            drain(p, pend[p])
            if p + 2 < NPASS:
                pend.append(fire(p + 2))

    return gather_k


_sc_gather = _make_sc_gather()


def _pad_body(t_ref, o_ref):
    xt = t_ref[...].T.astype(jnp.bfloat16)
    o_ref[...] = jnp.concatenate(
        [xt, jnp.zeros((xt.shape[0], EMBP - EMB), jnp.bfloat16)], axis=1)


def _pad_table(tableT, bm=8192):
    grid = (VOCAB + bm - 1) // bm
    return pl.pallas_call(
        _pad_body,
        grid=(grid,),
        in_specs=[pl.BlockSpec((EMB, bm), lambda i: (0, i))],
        out_specs=pl.BlockSpec((bm, EMBP), lambda i: (i, 0)),
        out_shape=jax.ShapeDtypeStruct((VOCAB, EMBP), jnp.bfloat16),
    )(tableT)


def _mlp_body(x_ref, w1_ref, b1_ref, w2_ref, b2_ref, o_ref):
    x = x_ref[...]
    h = jnp.tanh(
        lax.dot_general(x, w1_ref[...], (((1,), (1,)), ((), ())),
                        preferred_element_type=jnp.float32) + b1_ref[...])
    o_ref[...] = lax.dot_general(h, w2_ref[...], (((1,), (1,)), ((), ())),
                                 preferred_element_type=jnp.float32) + b2_ref[...]


def _mlp(x, W1p, b1, W2, b2, bm=2048):
    kp = WIN * EMBP
    return pl.pallas_call(
        _mlp_body,
        grid=(BATCH // bm,),
        in_specs=[
            pl.BlockSpec((bm, kp), lambda i: (i, 0)),
            pl.BlockSpec((100, kp), lambda i: (0, 0)),
            pl.BlockSpec((1, 100), lambda i: (0, 0)),
            pl.BlockSpec((36, 100), lambda i: (0, 0)),
            pl.BlockSpec((1, 36), lambda i: (0, 0)),
        ],
        out_specs=pl.BlockSpec((bm, 36), lambda i: (i, 0)),
        out_shape=jax.ShapeDtypeStruct((BATCH, 36), jnp.float32),
    )(x, W1p, b1.reshape(1, 100), W2, b2.reshape(1, 36))


def kernel(input, table, W1, b1, W2, b2):
    tablep = _pad_table(table.T)
    idx = input.reshape(NW, NCHUNK, CHUNK).astype(jnp.int32)
    rows = _sc_gather(tablep, idx)
    x = rows.reshape(BATCH, WIN * EMBP)
    W1p = jnp.pad(W1.reshape(100, WIN, EMB),
                  ((0, 0), (0, 0), (0, EMBP - EMB))
                  ).reshape(100, WIN * EMBP).astype(jnp.bfloat16)
    return _mlp(x, W1p, b1, W2, b2)


# speedup vs baseline: 3.2477x; 3.2477x over previous
import functools

import jax
import jax.numpy as jnp
from jax import lax
from jax.experimental import pallas as pl
from jax.experimental.pallas import tpu as pltpu
from jax.experimental.pallas import tpu_sc as plsc

VOCAB = 1000000
BATCH = 16384
WIN = 5
EMB = 50
EMBP = 128

NC = 2
NS = 16
NW = NC * NS
CHUNK = 128
TOTAL = BATCH * WIN
PER_W = TOTAL // NW
NCHUNK = PER_W // CHUNK
NPASS = 10
NBUF = NCHUNK // NPASS


def _make_sc_gather():
    mesh = plsc.VectorSubcoreMesh(core_axis_name="c", subcore_axis_name="s")

    @functools.partial(
        pl.kernel,
        mesh=mesh,
        compiler_params=pltpu.CompilerParams(use_tc_tiling_on_sc=True),
        out_type=jax.ShapeDtypeStruct((NW, NCHUNK, CHUNK, EMBP), jnp.float32),
        scratch_types=[
            pltpu.VMEM((NCHUNK, CHUNK), jnp.int32),
            pltpu.VMEM((NBUF, CHUNK, EMBP), jnp.float32),
            pltpu.VMEM((NBUF, CHUNK, EMBP), jnp.float32),
            pltpu.SemaphoreType.DMA,
            pltpu.SemaphoreType.DMA,
        ],
    )
    def gather_k(table_hbm, idx_hbm, out_hbm, idx_v, rows_a, rows_b, sem_a,
                 sem_b):
        wid = lax.axis_index("s") * NC + lax.axis_index("c")
        pltpu.sync_copy(idx_hbm.at[wid], idx_v)
        bufs = (rows_a, rows_b)
        sems = (sem_a, sem_b)

        def fire(p):
            buf, sem = bufs[p % 2], sems[p % 2]
            return [pltpu.async_copy(table_hbm.at[idx_v.at[p * NBUF + j]],
                                     buf.at[j], sem)
                    for j in range(NBUF)]

        def drain(p, copies):
            for c in copies:
                c.wait()
            pltpu.sync_copy(bufs[p % 2],
                            out_hbm.at[wid, pl.ds(p * NBUF, NBUF)])

        pend = [fire(0), fire(1)]
        for p in range(NPASS):
            drain(p, pend[p])
            if p + 2 < NPASS:
                pend.append(fire(p + 2))

    return gather_k


_sc_gather = _make_sc_gather()


def _pad_body(t_ref, o_ref):
    xt = t_ref[...].T
    o_ref[...] = jnp.concatenate(
        [xt, jnp.zeros((xt.shape[0], EMBP - EMB), jnp.float32)], axis=1)


def _pad_table(tableT, bm=16384):
    grid = (VOCAB + bm - 1) // bm
    return pl.pallas_call(
        _pad_body,
        grid=(grid,),
        in_specs=[pl.BlockSpec((EMB, bm), lambda i: (0, i))],
        out_specs=pl.BlockSpec((bm, EMBP), lambda i: (i, 0)),
        out_shape=jax.ShapeDtypeStruct((VOCAB, EMBP), jnp.float32),
    )(tableT)


def _mlp_body(x_ref, w1_ref, b1_ref, w2_ref, b2_ref, o_ref):
    x = x_ref[...]
    h = jnp.tanh(
        lax.dot_general(x, w1_ref[...], (((1,), (1,)), ((), ())),
                        preferred_element_type=jnp.float32) + b1_ref[...])
    o_ref[...] = lax.dot_general(h, w2_ref[...], (((1,), (1,)), ((), ())),
                                 preferred_element_type=jnp.float32) + b2_ref[...]


def _mlp(x, W1p, b1, W2, b2, bm=2048):
    kp = WIN * EMBP
    return pl.pallas_call(
        _mlp_body,
        grid=(BATCH // bm,),
        in_specs=[
            pl.BlockSpec((bm, kp), lambda i: (i, 0)),
            pl.BlockSpec((100, kp), lambda i: (0, 0)),
            pl.BlockSpec((1, 100), lambda i: (0, 0)),
            pl.BlockSpec((36, 100), lambda i: (0, 0)),
            pl.BlockSpec((1, 36), lambda i: (0, 0)),
        ],
        out_specs=pl.BlockSpec((bm, 36), lambda i: (i, 0)),
        out_shape=jax.ShapeDtypeStruct((BATCH, 36), jnp.float32),
    )(x, W1p, b1.reshape(1, 100), W2, b2.reshape(1, 36))


def kernel(input, table, W1, b1, W2, b2):
    tablep = _pad_table(table.T)
    idx = input.reshape(NW, NCHUNK, CHUNK).astype(jnp.int32)
    rows = _sc_gather(tablep, idx)
    x = rows.reshape(BATCH, WIN * EMBP)
    W1p = jnp.pad(W1.reshape(100, WIN, EMB),
                  ((0, 0), (0, 0), (0, EMBP - EMB))).reshape(100, WIN * EMBP)
    return _mlp(x, W1p, b1, W2, b2)


# speedup vs baseline: 3.3034x; 1.0171x over previous
import functools

import jax
import jax.numpy as jnp
from jax import lax
from jax.experimental import pallas as pl
from jax.experimental.pallas import tpu as pltpu
from jax.experimental.pallas import tpu_sc as plsc

VOCAB = 1000000
BATCH = 16384
WIN = 5
EMB = 50
EMBP = 128

NC = 2
NS = 16
NW = NC * NS
CHUNK = 128
TOTAL = BATCH * WIN
PER_W = TOTAL // NW
NCHUNK = PER_W // CHUNK
NPASS = 10
NBUF = NCHUNK // NPASS


def _make_sc_gather():
    mesh = plsc.VectorSubcoreMesh(core_axis_name="c", subcore_axis_name="s")

    @functools.partial(
        pl.kernel,
        mesh=mesh,
        compiler_params=pltpu.CompilerParams(use_tc_tiling_on_sc=True),
        out_type=jax.ShapeDtypeStruct((NW, NCHUNK, CHUNK, EMBP), jnp.float32),
        scratch_types=[
            pltpu.VMEM((NCHUNK, CHUNK), jnp.int32),
            pltpu.VMEM((NBUF, CHUNK, EMBP), jnp.float32),
            pltpu.VMEM((NBUF, CHUNK, EMBP), jnp.float32),
            pltpu.SemaphoreType.DMA,
            pltpu.SemaphoreType.DMA,
        ],
    )
    def gather_k(table_hbm, idx_hbm, out_hbm, idx_v, rows_a, rows_b, sem_a,
                 sem_b):
        wid = lax.axis_index("s") * NC + lax.axis_index("c")
        pltpu.sync_copy(idx_hbm.at[wid], idx_v)
        bufs = (rows_a, rows_b)
        sems = (sem_a, sem_b)

        def fire(p):
            buf, sem = bufs[p % 2], sems[p % 2]
            return [pltpu.async_copy(table_hbm.at[idx_v.at[p * NBUF + j]],
                                     buf.at[j], sem)
                    for j in range(NBUF)]

        def drain(p, copies):
            for c in copies:
                c.wait()
            pltpu.sync_copy(bufs[p % 2],
                            out_hbm.at[wid, pl.ds(p * NBUF, NBUF)])

        pend = [fire(0), fire(1)]
        for p in range(NPASS):
            drain(p, pend[p])
            if p + 2 < NPASS:
                pend.append(fire(p + 2))

    return gather_k


_sc_gather = _make_sc_gather()


def _pad_body(t_ref, o_ref):
    xt = t_ref[...].T
    o_ref[...] = jnp.concatenate(
        [xt, jnp.zeros((xt.shape[0], EMBP - EMB), jnp.float32)], axis=1)


def _pad_table(tableT, bm=32768):
    grid = (VOCAB + bm - 1) // bm
    return pl.pallas_call(
        _pad_body,
        grid=(grid,),
        in_specs=[pl.BlockSpec((EMB, bm), lambda i: (0, i))],
        out_specs=pl.BlockSpec((bm, EMBP), lambda i: (i, 0)),
        out_shape=jax.ShapeDtypeStruct((VOCAB, EMBP), jnp.float32),
    )(tableT)


def _mlp_body(x_ref, w1_ref, b1_ref, w2_ref, b2_ref, o_ref):
    x = x_ref[...]
    h = jnp.tanh(
        lax.dot_general(x, w1_ref[...], (((1,), (1,)), ((), ())),
                        preferred_element_type=jnp.float32) + b1_ref[...])
    o_ref[...] = lax.dot_general(h, w2_ref[...], (((1,), (1,)), ((), ())),
                                 preferred_element_type=jnp.float32) + b2_ref[...]


def _mlp(x, W1p, b1, W2, b2, bm=2048):
    kp = WIN * EMBP
    return pl.pallas_call(
        _mlp_body,
        grid=(BATCH // bm,),
        in_specs=[
            pl.BlockSpec((bm, kp), lambda i: (i, 0)),
            pl.BlockSpec((100, kp), lambda i: (0, 0)),
            pl.BlockSpec((1, 100), lambda i: (0, 0)),
            pl.BlockSpec((36, 100), lambda i: (0, 0)),
            pl.BlockSpec((1, 36), lambda i: (0, 0)),
        ],
        out_specs=pl.BlockSpec((bm, 36), lambda i: (i, 0)),
        out_shape=jax.ShapeDtypeStruct((BATCH, 36), jnp.float32),
    )(x, W1p, b1.reshape(1, 100), W2, b2.reshape(1, 36))


def kernel(input, table, W1, b1, W2, b2):
    tablep = _pad_table(table.T)
    idx = input.reshape(NW, NCHUNK, CHUNK).astype(jnp.int32)
    rows = _sc_gather(tablep, idx)
    x = rows.reshape(BATCH, WIN * EMBP)
    W1p = jnp.pad(W1.reshape(100, WIN, EMB),
                  ((0, 0), (0, 0), (0, EMBP - EMB))).reshape(100, WIN * EMBP)
    return _mlp(x, W1p, b1, W2, b2)
